# baseline (speedup 1.0000x reference)
import functools

import jax
import jax.numpy as jnp
from jax import lax
from jax.experimental import pallas as pl
from jax.experimental.pallas import tpu as pltpu
from jax.experimental.pallas import tpu_sc as plsc

N_PIX = 262144
D = 128
NC = 2
NS = 16
NW = NC * NS
BPW = N_PIX // NW
EC = 128
NEC = BPW // EC
WAVE = 8
C = 32
NCHUNK = BPW // C


def _sc_interpolate(vertex_features, f0, f1, f2, w0, w1, w2, face_ids):
    mesh = plsc.VectorSubcoreMesh(core_axis_name="c", subcore_axis_name="s")

    @functools.partial(
        pl.kernel,
        out_type=jax.ShapeDtypeStruct((N_PIX, D), jnp.float32),
        mesh=mesh,
        compiler_params=pltpu.CompilerParams(use_tc_tiling_on_sc=False),
        scratch_types=[
            pltpu.VMEM((BPW,), jnp.int32),
            pltpu.VMEM((BPW,), jnp.int32),
            pltpu.VMEM((BPW,), jnp.int32),
            pltpu.VMEM((BPW,), jnp.int32),
            pltpu.VMEM((BPW,), jnp.float32),
            pltpu.VMEM((BPW,), jnp.float32),
            pltpu.VMEM((BPW,), jnp.float32),
            pltpu.VMEM((C, D // 2), jnp.int32),
            pltpu.VMEM((C, D // 2), jnp.int32),
            pltpu.VMEM((C, D // 2), jnp.int32),
            pltpu.VMEM((C, D // 2), jnp.int32),
            pltpu.VMEM((C, D // 2), jnp.int32),
            pltpu.VMEM((C, D // 2), jnp.int32),
            pltpu.VMEM((C, D), jnp.float32),
            pltpu.VMEM((C, D), jnp.float32),
            pltpu.SemaphoreType.DMA,
            pltpu.SemaphoreType.DMA,
            pltpu.SemaphoreType.DMA,
            pltpu.SemaphoreType.DMA,
            pltpu.SemaphoreType.DMA,
        ],
    )
    def k(vf_hbm, f0_hbm, f1_hbm, f2_hbm, w0_hbm, w1_hbm, w2_hbm, fid_hbm,
          out_hbm, fid_b, i0_b, i1_b, i2_b, w0_b, w1_b, w2_b,
          r00, r01, r02, r10, r11, r12, o0, o1,
          sem_pre, sem_r0, sem_r1, sem_o0, sem_o1):
        wid = lax.axis_index("s") * NC + lax.axis_index("c")
        base = wid * BPW

        rbufs = ((r00, r01, r02), (r10, r11, r12))
        obufs = (o0, o1)
        rsems = (sem_r0, sem_r1)
        osems = (sem_o0, sem_o1)
        ftabs = (f0_hbm, f1_hbm, f2_hbm)
        itabs = (i0_b, i1_b, i2_b)
        wtabs = (w0_b, w1_b, w2_b)
        whtabs = (w0_hbm, w1_hbm, w2_hbm)

        pltpu.sync_copy(fid_hbm.at[pl.ds(base, BPW)], fid_b)
        for t in range(3):
            pltpu.sync_copy(whtabs[t].at[pl.ds(base, BPW)], wtabs[t])

        def wave(wv, c):
            fired = []
            for j in range(WAVE):
                off = (wv * WAVE + j) * EC
                s = pl.ds(off, EC)
                for t in range(3):
                    fired.append(pltpu.async_copy(
                        ftabs[t].at[fid_b.at[s]], itabs[t].at[s], sem_pre))
            for cp in fired:
                cp.wait()
            return c

        lax.fori_loop(0, NEC // WAVE, wave, 0)

        def fire_rows(g, st):
            s = pl.ds(g * C, C)
            for t in range(3):
                pltpu.async_copy(vf_hbm.at[itabs[t].at[s]], rbufs[st][t],
                                 rsems[st])

        def wait_rows(g, st):
            s = pl.ds(g * C, C)
            for t in range(3):
                pltpu.make_async_copy(vf_hbm.at[itabs[t].at[s]],
                                      rbufs[st][t], rsems[st]).wait()

        def fire_out(g, st):
            pltpu.async_copy(obufs[st], out_hbm.at[pl.ds(base + g * C, C)],
                             osems[st])

        def wait_out(g, st):
            pltpu.make_async_copy(obufs[st],
                                  out_hbm.at[pl.ds(base + g * C, C)],
                                  osems[st]).wait()

        def compute(g, st):
            r0, r1, r2 = rbufs[st]
            ov = obufs[st]
            lb = g * C
            hi_mask = jnp.int32(-65536)

            def expand(w):
                lo = lax.bitcast_convert_type(w << 16, jnp.float32)
                hi = lax.bitcast_convert_type(w & hi_mask, jnp.float32)
                return lo, hi

            for gi in range(C // 16):
                pb = gi * 16
                wv0 = w0_b[pl.ds(lb + pb, 16)]
                wv1 = w1_b[pl.ds(lb + pb, 16)]
                wv2 = w2_b[pl.ds(lb + pb, 16)]
                for j in range(16):
                    p = pb + j
                    a0 = wv0[j]
                    a1 = wv1[j]
                    a2 = wv2[j]
                    for q in range(D // 32):
                        s = pl.ds(q * 16, 16)
                        lo0, hi0 = expand(r0[p, s])
                        lo1, hi1 = expand(r1[p, s])
                        lo2, hi2 = expand(r2[p, s])
                        ov[p, pl.ds(q * 32, 16)] = (
                            a0 * lo0 + a1 * lo1 + a2 * lo2)
                        ov[p, pl.ds(q * 32 + 16, 16)] = (
                            a0 * hi0 + a1 * hi1 + a2 * hi2)

        fire_rows(0, 0)

        def pair(g2, c):
            g = 2 * g2
            wait_rows(g, 0)
            fire_rows(g + 1, 1)

            @pl.when(g2 >= 1)
            def _():
                wait_out(g - 2, 0)

            compute(g, 0)
            fire_out(g, 0)

            wait_rows(g + 1, 1)

            @pl.when(g2 <= NCHUNK // 2 - 2)
            def _():
                fire_rows(g + 2, 0)

            @pl.when(g2 >= 1)
            def _():
                wait_out(g - 1, 1)

            compute(g + 1, 1)
            fire_out(g + 1, 1)
            return c

        lax.fori_loop(0, NCHUNK // 2, pair, 0)
        wait_out(NCHUNK - 2, 0)
        wait_out(NCHUNK - 1, 1)

    return k(vertex_features, f0, f1, f2, w0, w1, w2, face_ids)


def _column_perm():
    perm = []
    for q in range(D // 32):
        for j in range(16):
            perm.append(32 * q + j)
            perm.append(32 * q + 16 + j)
    return jnp.asarray(perm, dtype=jnp.int32)


def kernel(vertex_features, faces, barycentric_coords, face_ids):
    faces = faces.astype(jnp.int32)
    face_ids = face_ids.astype(jnp.int32)
    vf16 = vertex_features.astype(jnp.bfloat16)[:, _column_perm()]
    vfi = lax.bitcast_convert_type(
        vf16.reshape(vertex_features.shape[0], D // 2, 2), jnp.int32)
    return _sc_interpolate(vfi, faces[:, 0], faces[:, 1],
                           faces[:, 2], barycentric_coords[:, 0],
                           barycentric_coords[:, 1],
                           barycentric_coords[:, 2], face_ids)
